# baseline (speedup 1.0000x reference)
import math

import jax
import jax.numpy as jnp
import numpy as np
from jax.experimental import pallas as pl
from jax.experimental.pallas import tpu as pltpu

N = 512
N_VEC = 4
N_FEAT = 32
D = 64
RB = 16
NW = RB * N

_DN = (((1,), (0,)), ((), ()))
_DNT = (((0,), (0,)), ((), ()))
_DNL = (((1,), (1,)), ((), ()))


def _egcl_kernel(pT_ref, featT_ref, AT_ref, BT_ref, CT_ref, We1T_ref,
                 Wx0iT_ref, Wx1T_ref, WtT_ref, MT_ref, SEG_ref,
                 mi_ref, vec_ref, fbT8_ref, rT_ref):
    i0 = pl.program_id(0)
    f32 = jnp.float32

    @pl.when(i0 == 0)
    def _precompute():
        fbT = jax.lax.dot_general(BT_ref[...], featT_ref[...], _DN,
                                  preferred_element_type=f32)
        fbT8_ref[...] = jnp.concatenate([fbT] * RB, axis=1)
        rT_ref[...] = jax.lax.dot_general(CT_ref[...], featT_ref[...], _DN,
                                          preferred_element_type=f32)

    lane = jax.lax.broadcasted_iota(jnp.int32, (RB, N), 1)
    sub = jax.lax.broadcasted_iota(jnp.int32, (RB, N), 0)
    OH = jnp.where(lane == RB * i0 + sub, 1.0, 0.0)
    PR = jax.lax.dot_general(pT_ref[...], OH, _DNL,
                             preferred_element_type=f32)
    BIAS = jax.lax.dot_general(rT_ref[...], OH, _DNL,
                               preferred_element_type=f32)

    pT = pT_ref[...]
    v = jnp.concatenate([pT - PR[:, r:r + 1] for r in range(RB)], axis=1)
    vx = v[0:4, :]
    vy = v[4:8, :]
    vz = v[8:12, :]
    len2 = vx * vx + vy * vy + vz * vz
    lengths = jnp.sqrt(len2)
    inv = jnp.where(len2 > 0, 1.0 / jnp.where(len2 > 0, lengths, 1.0), 0.0)

    bf16 = jnp.bfloat16
    LHS = jnp.concatenate([AT_ref[...], BIAS.astype(bf16)], axis=1)
    RHS = jnp.concatenate([lengths.astype(bf16), SEG_ref[...]], axis=0)
    h0 = jax.lax.dot_general(LHS, RHS, _DN, preferred_element_type=f32)
    h0 = jax.nn.silu(h0 + fbT8_ref[...])
    m = jax.nn.silu(jax.lax.dot_general(We1T_ref[...], h0.astype(bf16), _DN,
                                        preferred_element_type=f32))

    mb = m.astype(bf16)
    hx_e = jax.lax.dot_general(Wx0iT_ref[...], mb, _DN,
                               preferred_element_type=f32)
    e = jax.nn.sigmoid(hx_e[D:D + 1, :])
    glane = jax.lax.broadcasted_iota(jnp.int32, (1, NW), 1)
    seg = glane // N
    e = jnp.where(glane == 513 * seg + RB * i0, 0.0, e)
    E = e * SEG_ref[...]
    mi_ref[0] = jax.lax.dot_general(E, m, _DNL,
                                    preferred_element_type=f32)

    hx = jax.nn.silu(hx_e[0:D, :])
    phi = jax.nn.silu(jax.lax.dot_general(Wx1T_ref[...], hx.astype(bf16), _DN,
                                          preferred_element_type=f32))
    T = jax.lax.dot_general(WtT_ref[...], phi.astype(bf16), _DN,
                            preferred_element_type=f32)

    ux = vx * inv
    uy = vy * inv
    uz = vz * inv
    P = jnp.concatenate(
        [T * jnp.concatenate([ux, ux, ux, ux], axis=0),
         T * jnp.concatenate([uy, uy, uy, uy], axis=0),
         T * jnp.concatenate([uz, uz, uz, uz], axis=0)], axis=0)

    out12T = jax.lax.dot_general(MT_ref[...], P, _DN,
                                 preferred_element_type=f32)
    for r in range(RB):
        out_r = pltpu.roll(out12T[:, r * N:(r + 1) * N],
                           N - 1 - RB * i0 - r, axis=1)
        vec_ref[r] = out_r[:, 0:N - 1]


def kernel(positions, features, W_e0, W_e1, W_inf, W_x0, W_x1, W_tp):
    f32 = jnp.float32

    pT = positions.transpose(2, 1, 0).reshape(3 * N_VEC, N)
    featT = features.T

    s0 = 1.0 / math.sqrt(N_VEC + 2 * N_FEAT)
    AT = W_e0[0:N_VEC].T * s0
    BT = W_e0[N_VEC:N_VEC + N_FEAT].T * s0
    CT = W_e0[N_VEC + N_FEAT:].T * s0
    sD = 1.0 / math.sqrt(D)
    We1T = W_e1.T * sD
    Wx0iT = jnp.concatenate([W_x0.T, W_inf.T], axis=0) * sD
    Wx1T = W_x1.T * sD
    WtT = (W_tp.transpose(2, 1, 0).reshape(N_VEC * N_VEC, D)
           * (math.sqrt(3.0) / math.sqrt(D * N_VEC)))
    Mnp = np.zeros((3 * N_VEC * N_VEC, 3 * N_VEC), dtype=np.float32)
    for c in range(3):
        for u in range(N_VEC):
            for k in range(N_VEC):
                Mnp[c * 16 + 4 * u + k, 3 * u + c] = 1.0
    MT = jnp.asarray(Mnp.T)
    bf16 = jnp.bfloat16
    AT = AT.astype(bf16)
    We1T = We1T.astype(bf16)
    Wx0iT = Wx0iT.astype(bf16)
    Wx1T = Wx1T.astype(bf16)
    WtT = WtT.astype(bf16)
    Snp = np.zeros((RB, NW), dtype=np.float32)
    for r in range(RB):
        Snp[r, r * N:(r + 1) * N] = 1.0
    SEG = jnp.asarray(Snp).astype(jnp.bfloat16)

    full = lambda shape: pl.BlockSpec(shape, lambda i: (0,) * len(shape))
    mi3, vec = pl.pallas_call(
        _egcl_kernel,
        grid=(N // RB,),
        in_specs=[
            full((3 * N_VEC, N)),
            full((N_FEAT, N)),
            full((D, N_VEC)),
            full((D, N_FEAT)),
            full((D, N_FEAT)),
            full((D, D)),
            full((D + 1, D)),
            full((D, D)),
            full((N_VEC * N_VEC, D)),
            full((3 * N_VEC, 3 * N_VEC * N_VEC)),
            full((RB, NW)),
        ],
        out_specs=[
            pl.BlockSpec((1, RB, D), lambda i: (i, 0, 0)),
            pl.BlockSpec((RB, 3 * N_VEC, N - 1), lambda i: (i, 0, 0)),
        ],
        out_shape=[
            jax.ShapeDtypeStruct((N // RB, RB, D), f32),
            jax.ShapeDtypeStruct((N, 3 * N_VEC, N - 1), f32),
        ],
        scratch_shapes=[
            pltpu.VMEM((D, NW), f32),
            pltpu.VMEM((D, N), f32),
        ],
    )(pT, featT, AT, BT, CT, We1T, Wx0iT, Wx1T, WtT, MT, SEG)

    m_i = mi3.reshape(N, D)
    vec_out = vec.transpose(0, 2, 1).reshape(N, N - 1, N_VEC, 3)
    return m_i, vec_out


# speedup vs baseline: 1.0112x; 1.0112x over previous
import math

import jax
import jax.numpy as jnp
import numpy as np
from jax.experimental import pallas as pl
from jax.experimental.pallas import tpu as pltpu

N = 512
N_VEC = 4
N_FEAT = 32
D = 64
RB = 16
NW = RB * N

_DN = (((1,), (0,)), ((), ()))
_DNT = (((0,), (0,)), ((), ()))
_DNL = (((1,), (1,)), ((), ()))


def _egcl_kernel(pT_ref, featT_ref, AT_ref, BT_ref, CT_ref, We1T_ref,
                 Wx0iT_ref, Wx1T_ref, WtT_ref, MT_ref, SEG_ref,
                 mi_ref, vec_ref, fbT8_ref, rT_ref):
    i0 = pl.program_id(0)
    f32 = jnp.float32

    @pl.when(i0 == 0)
    def _precompute():
        fbT = jax.lax.dot_general(BT_ref[...], featT_ref[...], _DN,
                                  preferred_element_type=f32)
        fbT8_ref[...] = jnp.concatenate([fbT] * RB, axis=1)
        rT_ref[...] = jax.lax.dot_general(CT_ref[...], featT_ref[...], _DN,
                                          preferred_element_type=f32)

    lane = jax.lax.broadcasted_iota(jnp.int32, (RB, N), 1)
    sub = jax.lax.broadcasted_iota(jnp.int32, (RB, N), 0)
    OH = jnp.where(lane == RB * i0 + sub, 1.0, 0.0)
    PR = jax.lax.dot_general(pT_ref[...], OH, _DNL,
                             preferred_element_type=f32)
    BIAS = jax.lax.dot_general(rT_ref[...], OH, _DNL,
                               preferred_element_type=f32)

    pT = pT_ref[...]
    v = jnp.concatenate([pT - PR[:, r:r + 1] for r in range(RB)], axis=1)
    vx = v[0:4, :]
    vy = v[4:8, :]
    vz = v[8:12, :]
    len2 = vx * vx + vy * vy + vz * vz
    lengths = jnp.sqrt(len2)
    inv = jnp.where(len2 > 0, 1.0 / jnp.where(len2 > 0, lengths, 1.0), 0.0)

    LHS = jnp.concatenate([AT_ref[...], BIAS], axis=1)
    RHS = jnp.concatenate([lengths, SEG_ref[...]], axis=0)
    h0 = jax.lax.dot_general(LHS, RHS, _DN, preferred_element_type=f32)
    h0 = jax.nn.silu(h0 + fbT8_ref[...])
    m = jax.nn.silu(jax.lax.dot_general(We1T_ref[...], h0, _DN,
                                        preferred_element_type=f32))

    hx_e = jax.lax.dot_general(Wx0iT_ref[...], m, _DN,
                               preferred_element_type=f32)
    e = jax.nn.sigmoid(hx_e[D:D + 1, :])
    glane = jax.lax.broadcasted_iota(jnp.int32, (1, NW), 1)
    seg = glane // N
    e = jnp.where(glane == 513 * seg + RB * i0, 0.0, e)
    E = e * SEG_ref[...]
    mi_ref[0] = jax.lax.dot_general(E, m, _DNL,
                                    preferred_element_type=f32)

    hx = jax.nn.silu(hx_e[0:D, :])
    phi = jax.nn.silu(jax.lax.dot_general(Wx1T_ref[...], hx, _DN,
                                          preferred_element_type=f32))
    T = jax.lax.dot_general(WtT_ref[...], phi, _DN,
                            preferred_element_type=f32)

    ux = vx * inv
    uy = vy * inv
    uz = vz * inv
    P = jnp.concatenate(
        [T * jnp.concatenate([ux, ux, ux, ux], axis=0),
         T * jnp.concatenate([uy, uy, uy, uy], axis=0),
         T * jnp.concatenate([uz, uz, uz, uz], axis=0)], axis=0)

    out12T = jax.lax.dot_general(MT_ref[...], P, _DN,
                                 preferred_element_type=f32)
    for r in range(RB):
        out_r = pltpu.roll(out12T[:, r * N:(r + 1) * N],
                           N - 1 - RB * i0 - r, axis=1)
        vec_ref[r] = out_r[:, 0:N - 1]


def kernel(positions, features, W_e0, W_e1, W_inf, W_x0, W_x1, W_tp):
    f32 = jnp.float32

    pT = positions.transpose(2, 1, 0).reshape(3 * N_VEC, N)
    featT = features.T

    s0 = 1.0 / math.sqrt(N_VEC + 2 * N_FEAT)
    AT = W_e0[0:N_VEC].T * s0
    BT = W_e0[N_VEC:N_VEC + N_FEAT].T * s0
    CT = W_e0[N_VEC + N_FEAT:].T * s0
    sD = 1.0 / math.sqrt(D)
    We1T = W_e1.T * sD
    Wx0iT = jnp.concatenate([W_x0.T, W_inf.T], axis=0) * sD
    Wx1T = W_x1.T * sD
    WtT = (W_tp.transpose(2, 1, 0).reshape(N_VEC * N_VEC, D)
           * (math.sqrt(3.0) / math.sqrt(D * N_VEC)))
    Mnp = np.zeros((3 * N_VEC * N_VEC, 3 * N_VEC), dtype=np.float32)
    for c in range(3):
        for u in range(N_VEC):
            for k in range(N_VEC):
                Mnp[c * 16 + 4 * u + k, 3 * u + c] = 1.0
    MT = jnp.asarray(Mnp.T)
    Snp = np.zeros((RB, NW), dtype=np.float32)
    for r in range(RB):
        Snp[r, r * N:(r + 1) * N] = 1.0
    SEG = jnp.asarray(Snp)

    full = lambda shape: pl.BlockSpec(shape, lambda i: (0,) * len(shape))
    mi3, vec = pl.pallas_call(
        _egcl_kernel,
        grid=(N // RB,),
        in_specs=[
            full((3 * N_VEC, N)),
            full((N_FEAT, N)),
            full((D, N_VEC)),
            full((D, N_FEAT)),
            full((D, N_FEAT)),
            full((D, D)),
            full((D + 1, D)),
            full((D, D)),
            full((N_VEC * N_VEC, D)),
            full((3 * N_VEC, 3 * N_VEC * N_VEC)),
            full((RB, NW)),
        ],
        out_specs=[
            pl.BlockSpec((1, RB, D), lambda i: (i, 0, 0)),
            pl.BlockSpec((RB, 3 * N_VEC, N - 1), lambda i: (i, 0, 0)),
        ],
        out_shape=[
            jax.ShapeDtypeStruct((N // RB, RB, D), f32),
            jax.ShapeDtypeStruct((N, 3 * N_VEC, N - 1), f32),
        ],
        scratch_shapes=[
            pltpu.VMEM((D, NW), f32),
            pltpu.VMEM((D, N), f32),
        ],
    )(pT, featT, AT, BT, CT, We1T, Wx0iT, Wx1T, WtT, MT, SEG)

    m_i = mi3.reshape(N, D)
    vec_out = vec.transpose(0, 2, 1).reshape(N, N - 1, N_VEC, 3)
    return m_i, vec_out
